# baseline (speedup 1.0000x reference)
import functools
import math

import numpy as np

import jax
import jax.numpy as jnp
from jax import lax
from jax.experimental import pallas as pl
from jax.experimental.pallas import tpu as pltpu
from jax.experimental.pallas import tpu_sc as plsc

_BNS = 1.0 / math.sqrt(1.0 + 1e-3)
_SPATIAL = (25, 64, 64)
_B = 2
_YB = 8


def _pcall(body, out_shape, scratch_shapes=(), interpret=False):
    return pl.pallas_call(body, out_shape=out_shape,
                          scratch_shapes=list(scratch_shapes),
                          interpret=interpret)


_NSITE = _B * 25 * 64 * 64
_TRASH = 128
_DR = 16
_NSH = _NSITE // 2
_HALF = _NSH + _TRASH


def _sc_scatter(idx0, idx1, feats16):
    NP = idx0.shape[0]
    ZB = 128
    CH = 512
    GR = 128
    n_chunks = NP // CH
    n_groups = CH // GR
    rows_t = _NSH // 16
    nz_full, nz_rem = divmod(rows_t, ZB)

    mesh = plsc.VectorSubcoreMesh(core_axis_name="c", subcore_axis_name="s")
    scratch = ([pltpu.VMEM_SHARED((_HALF, _DR), jnp.float32),
                pltpu.VMEM((ZB, _DR), jnp.float32)]
               + [pltpu.VMEM((CH,), jnp.int32) for _ in range(2)]
               + [pltpu.VMEM((CH, _DR), jnp.float32) for _ in range(2)]
               + [pltpu.SemaphoreType.DMA for _ in range(4)])

    @functools.partial(pl.kernel, mesh=mesh,
                       out_type=jax.ShapeDtypeStruct((2 * _NSH, _DR),
                                                     jnp.float32),
                       scratch_types=scratch,
                       compiler_params=pltpu.CompilerParams(
                           use_tc_tiling_on_sc=False))
    def run(idx0_hbm, idx1_hbm, feats_hbm, out_hbm, shared, zbuf,
            ib0, ib1, rv0, rv1, zsem, l0sem, l1sem, ssem):
        idx_bufs = (ib0, ib1)
        rows_bufs = (rv0, rv1)
        load_sems = (l0sem, l1sem)
        cid = lax.axis_index("c")
        sid = lax.axis_index("s")

        def zrow(r, carry):
            zbuf[r] = jnp.zeros((_DR,), jnp.float32)
            return carry
        lax.fori_loop(0, ZB, zrow, 0)
        base = sid * rows_t
        zcopies = []
        for k in range(nz_full):
            zcopies.append(pltpu.async_copy(
                zbuf, shared.at[pl.ds(base + k * ZB, ZB)], zsem))
        if nz_rem:
            zcopies.append(pltpu.async_copy(
                zbuf.at[pl.ds(0, nz_rem)],
                shared.at[pl.ds(base + nz_full * ZB, nz_rem)], zsem))
        for cp in zcopies:
            cp.wait()
        plsc.subcore_barrier()

        for c in range(2):
            @pl.when(jnp.logical_and(cid == c, sid == 0))
            def _scatter_phase(c=c):
                ih = idx0_hbm if c == 0 else idx1_hbm

                def load(ch, sl):
                    return (pltpu.async_copy(
                                feats_hbm.at[pl.ds(ch * CH, CH)],
                                rows_bufs[sl], load_sems[sl]),
                            pltpu.async_copy(
                                ih.at[pl.ds(ch * CH, CH)],
                                idx_bufs[sl], load_sems[sl]))

                pend = load(0, 0)
                for ch in range(n_chunks):
                    sl = ch % 2
                    for cp in pend:
                        cp.wait()
                    if ch + 1 < n_chunks:
                        pend = load(ch + 1, 1 - sl)
                    pltpu.async_copy(rows_bufs[sl],
                                     shared.at[idx_bufs[sl]], ssem).wait()
        plsc.subcore_barrier()
        pltpu.sync_copy(shared.at[pl.ds(base, rows_t)],
                        out_hbm.at[pl.ds(cid * _NSH + base, rows_t)])

    return run(idx0, idx1, feats16)


def _fold_w(w, gi, go, sy, py, scale=1.0):
    tz, ty, tx, ci, co = w.shape
    P = np.zeros((3, gi, go, ty), np.float32)
    for si, s in enumerate((-1, 0, 1)):
        for g_in in range(gi):
            for g_out in range(go):
                dy = g_in + s * gi - sy * g_out + py
                if 0 <= dy < ty:
                    P[si, g_in, g_out, dy] = scale
    wf = jnp.einsum('sghy,zyxio->szxgiho', P, w)
    wf = wf.reshape(3, tz, tx, gi * ci, go * co).astype(jnp.bfloat16)
    s_active = [si for si in range(3) if P[si].any()]
    return wf, s_active


def _fold_w_stack(ws, gi, go, sy, py):
    nw, tz, ty, tx, ci, co = ws.shape
    P = np.zeros((3, gi, go, ty), np.float32)
    for si, s in enumerate((-1, 0, 1)):
        for g_in in range(gi):
            for g_out in range(go):
                dy = g_in + s * gi - sy * g_out + py
                if 0 <= dy < ty:
                    P[si, g_in, g_out, dy] = 1.0
    wf = jnp.einsum('sghy,nzyxio->nszxgiho', P, ws)
    wf = wf.reshape(nw, 3, tz, tx, gi * ci, go * co).astype(jnp.bfloat16)
    s_active = [si for si in range(3) if P[si].any()]
    return wf, s_active


def _taps(xp_ref, wf_ref, s_active, sz, sx, zo, xo, bz, bx, m, k, n, tz, tx):
    n_taps = len(s_active) * tz * tx
    s0 = s_active[0]

    def tap_body(t, acc):
        j = t // (tz * tx)
        dz = (t // tx) % tz
        dx = t % tx
        si = j + s0
        xs = xp_ref[pl.ds(dz + bz, sz * zo), pl.ds(si, _YB),
                    pl.ds(dx + bx, sx * xo), :]
        if sz > 1 or sx > 1:
            xs = xs.reshape(zo, sz, _YB, xo, sx, k)[:, 0, :, :, 0, :]
        return acc + jnp.dot(xs.reshape(m, k).astype(jnp.bfloat16),
                             wf_ref[si, dz, dx],
                             preferred_element_type=jnp.float32)

    return lax.fori_loop(0, n_taps, tap_body, jnp.zeros((m, n), jnp.float32))


def _store_padded(out_ref, val5, zo, xo, zr=2):
    z = jnp.float32(0.0)
    out_ref[pl.ds(0, 1)] = jnp.broadcast_to(z, out_ref.shape)[0:1]
    out_ref[pl.ds(zo + 1, zr)] = jnp.broadcast_to(z, out_ref.shape)[:zr]
    out_ref[:, pl.ds(0, 1)] = jnp.broadcast_to(z, out_ref.shape)[:, 0:1]
    out_ref[:, pl.ds(_YB + 1, 1)] = jnp.broadcast_to(z, out_ref.shape)[:, 0:1]
    out_ref[:, :, pl.ds(0, 1)] = jnp.broadcast_to(z, out_ref.shape)[:, :, 0:1]
    out_ref[:, :, pl.ds(xo + 1, 1)] = jnp.broadcast_to(
        z, out_ref.shape)[:, :, 0:1]
    out_ref[pl.ds(1, zo), pl.ds(1, _YB), pl.ds(1, xo), :] = val5


def _conv_core(src_ref, wf_ref, sa, sz, sx, zo, xo, bz=0, bx=0):
    _, tz, tx, k, n = wf_ref.shape
    m = zo * _YB * xo
    return _taps(src_ref, wf_ref, sa, sz, sx, zo, xo, bz, bx,
                 m, k, n, tz, tx)


def _interior(ref, zo, xo):
    return ref[pl.ds(1, zo), pl.ds(1, _YB), pl.ds(1, xo), :]


def _level1(xp, maskp, w_in_f, sa_in, wblk, sa1):
    zo, xo = 25, 64
    m = zo * _YB * xo

    def body(xp_ref, mask_ref, win_ref, wstk_ref, out_ref, scr_ref):
        mi = mask_ref[...].reshape(m, 128)

        acc = _conv_core(xp_ref, win_ref, sa_in, 1, 1, zo, xo)
        _store_padded(out_ref, jnp.maximum(acc * mi * _BNS, 0.0)
                      .reshape(zo, _YB, xo, 128), zo, xo)
        for wc1, wc2 in ((wstk_ref.at[0], wstk_ref.at[1]),
                         (wstk_ref.at[2], wstk_ref.at[3])):
            acc = _conv_core(out_ref, wc1, sa1, 1, 1, zo, xo)
            _store_padded(scr_ref, jnp.maximum(acc * mi * _BNS, 0.0)
                          .reshape(zo, _YB, xo, 128), zo, xo, zr=1)
            acc = _conv_core(scr_ref, wc2, sa1, 1, 1, zo, xo)
            ident = _interior(out_ref, zo, xo).reshape(m, 128)
            _store_padded(out_ref,
                          jnp.maximum(acc * mi * _BNS + ident, 0.0)
                          .reshape(zo, _YB, xo, 128), zo, xo)

    shape = (zo + 3, _YB + 2, xo + 2, 128)
    sshape = (zo + 2, _YB + 2, xo + 2, 128)
    return _pcall(body, jax.ShapeDtypeStruct(shape, jnp.float32),
                  scratch_shapes=[pltpu.VMEM(sshape, jnp.float32)])(
                      xp, maskp, w_in_f, wblk)


def _levels234(h1, maskp, wd, dil, wblk, wout, sa_out):
    dims = [(13, 32), (7, 16), (3, 8)]

    def body(*refs):
        (h1_ref, m1_ref, wd2_ref, wd3_ref, wd4_ref, dl2_ref, dl3_ref,
         dl4_ref, ws2_ref, ws3_ref, ws4_ref, wout_ref, out_ref,
         h2_ref, s2_ref, m2_ref, h3_ref, s3_ref, m3_ref,
         h4_ref, s4_ref, m4_ref) = refs

        lvl = [
            (h1_ref, m1_ref, wd2_ref, dl2_ref,
             tuple(ws2_ref.at[i] for i in range(4)),
             h2_ref, s2_ref, m2_ref, dims[0], 0),
            (h2_ref, m2_ref, wd3_ref, dl3_ref,
             tuple(ws3_ref.at[i] for i in range(4)),
             h3_ref, s3_ref, m3_ref, dims[1], 0),
            (h3_ref, m3_ref, wd4_ref, dl4_ref,
             tuple(ws4_ref.at[i] for i in range(4)),
             h4_ref, s4_ref, m4_ref, dims[2], 1),
        ]
        for (hin, min_, wd_ref, dl_ref, (w1, w2, w3, w4), hout, scr, mout,
             (zo, xo), bz) in lvl:
            m = zo * _YB * xo
            acc = _conv_core(hin, wd_ref, [0, 1], 2, 2, zo, xo, bz=bz)
            _store_padded(hout, jnp.maximum(acc * _BNS, 0.0)
                          .reshape(zo, _YB, xo, 128), zo, xo)
            macc = _conv_core(min_, dl_ref, [0, 1], 2, 2, zo, xo, bz=bz)
            _store_padded(mout, (macc > 0.0).astype(jnp.float32)
                          .reshape(zo, _YB, xo, 128), zo, xo)
            mi = _interior(mout, zo, xo).reshape(m, 128)
            for wc1, wc2 in ((w1, w2), (w3, w4)):
                acc = _conv_core(hout, wc1, [0, 1, 2], 1, 1, zo, xo)
                _store_padded(scr, jnp.maximum(acc * mi * _BNS, 0.0)
                              .reshape(zo, _YB, xo, 128), zo, xo)
                acc = _conv_core(scr, wc2, [0, 1, 2], 1, 1, zo, xo)
                ident = _interior(hout, zo, xo).reshape(m, 128)
                _store_padded(hout,
                              jnp.maximum(acc * mi * _BNS + ident, 0.0)
                              .reshape(zo, _YB, xo, 128), zo, xo)

        acc = _conv_core(h4_ref, wout_ref, sa_out, 2, 1, 1, 8, bz=1, bx=1)
        out_ref[...] = jnp.maximum(acc * _BNS, 0.0).reshape(1, _YB, 8, 128)

    def pbuf(zo, xo):
        return (zo + 3, _YB + 2, xo + 2, 128)

    scratch = []
    for zo, xo in dims:
        scratch += [pltpu.VMEM(pbuf(zo, xo), jnp.float32)] * 3
    return _pcall(body, jax.ShapeDtypeStruct((1, _YB, 8, 128), jnp.float32),
                  scratch_shapes=scratch)(
                      h1, maskp, wd[0], wd[1], wd[2], dil[0], dil[1], dil[2],
                      wblk[0], wblk[1], wblk[2], wout)


def kernel(voxel_features, voxel_coords, batch_size, params):
    Z, Y, X = _SPATIAL
    N = voxel_features.shape[0]
    p = params

    b = voxel_coords[:, 0] % batch_size
    z = voxel_coords[:, 1] % Z
    y = voxel_coords[:, 2] % Y
    x = voxel_coords[:, 3] % X

    feats16 = jnp.concatenate(
        [voxel_features,
         jnp.ones((N, 1), jnp.float32),
         jnp.zeros((N, _DR - 6), jnp.float32)], axis=1)
    idx = ((((b * Z + z) * _YB + y // 8) * X + x) * 8 + y % 8).astype(jnp.int32)
    npad = (-N) % 1024
    pad_i = jnp.full((npad,), _NSH, jnp.int32)
    idx0 = jnp.concatenate(
        [jnp.where(idx < _NSH, idx, _NSH).astype(jnp.int32), pad_i])
    idx1 = jnp.concatenate(
        [jnp.where(idx >= _NSH, idx - _NSH, _NSH).astype(jnp.int32), pad_i])
    feats_p = jnp.concatenate([feats16, jnp.zeros((npad, _DR), jnp.float32)])
    table = _sc_scatter(idx0, idx1, feats_p)
    dense8f = table.reshape(_B, Z, _YB, X, 128)
    mask1x = jnp.repeat(dense8f[..., 5::16], 16, axis=-1)
    mask1xp = jnp.pad(mask1x, ((0, 0), (1, 2), (1, 1), (1, 1), (0, 0)))

    w_in16 = jnp.pad(p['w_in'], ((0, 0), (0, 0), (0, 0), (0, 11), (0, 0)))
    w_in_f, sa_in = _fold_w(w_in16, 8, 8, 1, 1)

    wblk1, sa1 = _fold_w_stack(
        jnp.stack([p['r1a1'], p['r1a2'], p['r1b1'], p['r1b2']]), 8, 8, 1, 1)
    ws2, _ = _fold_w_stack(
        jnp.stack([p['r2a1'], p['r2a2'], p['r2b1'], p['r2b2']]), 4, 4, 1, 1)
    ws3, _ = _fold_w_stack(
        jnp.stack([p['r3a1'], p['r3a2'], p['r3b1'], p['r3b2']]), 2, 2, 1, 1)
    ws4, _ = _fold_w_stack(
        jnp.stack([p['r4a1'], p['r4a2'], p['r4b1'], p['r4b2']]), 1, 1, 1, 1)
    wblk234 = (ws2, ws3, ws4)
    wd2, _ = _fold_w(p['w_d2'], 8, 4, 2, 1)
    wd3, _ = _fold_w(p['w_d3'], 4, 2, 2, 1)
    wd4, _ = _fold_w(p['w_d4'], 2, 1, 2, 1)
    wout, sa_out = _fold_w(p['w_out'], 1, 1, 1, 0)
    dil2, _ = _fold_w(jnp.ones((3, 3, 3, 16, 32), jnp.float32), 8, 4, 2, 1)
    dil3, _ = _fold_w(jnp.ones((3, 3, 3, 32, 64), jnp.float32), 4, 2, 2, 1)
    dil4, _ = _fold_w(jnp.ones((3, 3, 3, 64, 128), jnp.float32), 2, 1, 2, 1)

    outs = []
    for bi in range(_B):
        xp0 = jnp.pad(dense8f[bi], ((1, 2), (1, 1), (1, 1), (0, 0)))
        h1 = _level1(xp0, mask1x[bi], w_in_f, sa_in, wblk1, sa1)
        out = _levels234(h1, mask1xp[bi], (wd2, wd3, wd4),
                         (dil2, dil3, dil4), wblk234, wout, sa_out)
        outs.append(out)

    return jnp.stack(outs)


# speedup vs baseline: 1.0078x; 1.0078x over previous
import functools
import math

import numpy as np

import jax
import jax.numpy as jnp
from jax import lax
from jax.experimental import pallas as pl
from jax.experimental.pallas import tpu as pltpu
from jax.experimental.pallas import tpu_sc as plsc

_BNS = 1.0 / math.sqrt(1.0 + 1e-3)
_SPATIAL = (25, 64, 64)
_B = 2
_YB = 8


def _pcall(body, out_shape, scratch_shapes=(), interpret=False):
    return pl.pallas_call(body, out_shape=out_shape,
                          scratch_shapes=list(scratch_shapes),
                          interpret=interpret)


_NSITE = _B * 25 * 64 * 64
_TRASH = 128
_DR = 16
_NSH = _NSITE // 2
_HALF = _NSH + _TRASH


def _sc_scatter(idx0, idx1, feats16):
    NP = idx0.shape[0]
    ZB = 128
    CH = 512
    GR = 128
    n_chunks = NP // CH
    n_groups = CH // GR
    rows_t = _NSH // 16
    nz_full, nz_rem = divmod(rows_t, ZB)

    mesh = plsc.VectorSubcoreMesh(core_axis_name="c", subcore_axis_name="s")
    scratch = ([pltpu.VMEM_SHARED((_HALF, _DR), jnp.float32),
                pltpu.VMEM((ZB, _DR), jnp.float32)]
               + [pltpu.VMEM((CH,), jnp.int32) for _ in range(2)]
               + [pltpu.VMEM((CH, _DR), jnp.float32) for _ in range(2)]
               + [pltpu.SemaphoreType.DMA for _ in range(4)])

    @functools.partial(pl.kernel, mesh=mesh,
                       out_type=jax.ShapeDtypeStruct((2 * _NSH, _DR),
                                                     jnp.float32),
                       scratch_types=scratch,
                       compiler_params=pltpu.CompilerParams(
                           use_tc_tiling_on_sc=False))
    def run(idx0_hbm, idx1_hbm, feats_hbm, out_hbm, shared, zbuf,
            ib0, ib1, rv0, rv1, zsem, l0sem, l1sem, ssem):
        idx_bufs = (ib0, ib1)
        rows_bufs = (rv0, rv1)
        load_sems = (l0sem, l1sem)
        cid = lax.axis_index("c")
        sid = lax.axis_index("s")

        def zrow(r, carry):
            zbuf[r] = jnp.zeros((_DR,), jnp.float32)
            return carry
        lax.fori_loop(0, ZB, zrow, 0)
        base = sid * rows_t
        zcopies = []
        for k in range(nz_full):
            zcopies.append(pltpu.async_copy(
                zbuf, shared.at[pl.ds(base + k * ZB, ZB)], zsem))
        if nz_rem:
            zcopies.append(pltpu.async_copy(
                zbuf.at[pl.ds(0, nz_rem)],
                shared.at[pl.ds(base + nz_full * ZB, nz_rem)], zsem))
        for cp in zcopies:
            cp.wait()
        plsc.subcore_barrier()

        for c in range(2):
            @pl.when(jnp.logical_and(cid == c, sid == 0))
            def _scatter_phase(c=c):
                ih = idx0_hbm if c == 0 else idx1_hbm

                def load(ch, sl):
                    return (pltpu.async_copy(
                                feats_hbm.at[pl.ds(ch * CH, CH)],
                                rows_bufs[sl], load_sems[sl]),
                            pltpu.async_copy(
                                ih.at[pl.ds(ch * CH, CH)],
                                idx_bufs[sl], load_sems[sl]))

                pend = load(0, 0)
                for ch in range(n_chunks):
                    sl = ch % 2
                    for cp in pend:
                        cp.wait()
                    if ch + 1 < n_chunks:
                        pend = load(ch + 1, 1 - sl)
                    pltpu.async_copy(rows_bufs[sl],
                                     shared.at[idx_bufs[sl]], ssem).wait()
        plsc.subcore_barrier()
        pltpu.sync_copy(shared.at[pl.ds(base, rows_t)],
                        out_hbm.at[pl.ds(cid * _NSH + base, rows_t)])

    return run(idx0, idx1, feats16)


def _fold_w(w, gi, go, sy, py, scale=1.0):
    tz, ty, tx, ci, co = w.shape
    P = np.zeros((3, gi, go, ty), np.float32)
    for si, s in enumerate((-1, 0, 1)):
        for g_in in range(gi):
            for g_out in range(go):
                dy = g_in + s * gi - sy * g_out + py
                if 0 <= dy < ty:
                    P[si, g_in, g_out, dy] = scale
    wf = jnp.einsum('sghy,zyxio->szxgiho', P, w)
    wf = wf.reshape(3, tz, tx, gi * ci, go * co).astype(jnp.bfloat16)
    s_active = [si for si in range(3) if P[si].any()]
    return wf, s_active


def _taps(xp_ref, wf_ref, s_active, sz, sx, zo, xo, bz, bx, m, k, n, tz, tx):
    n_taps = len(s_active) * tz * tx
    s0 = s_active[0]

    def tap_body(t, acc):
        j = t // (tz * tx)
        dz = (t // tx) % tz
        dx = t % tx
        si = j + s0
        xs = xp_ref[pl.ds(dz + bz, sz * zo), pl.ds(si, _YB),
                    pl.ds(dx + bx, sx * xo), :]
        if sz > 1 or sx > 1:
            xs = xs.reshape(zo, sz, _YB, xo, sx, k)[:, 0, :, :, 0, :]
        return acc + jnp.dot(xs.reshape(m, k).astype(jnp.bfloat16),
                             wf_ref[si, dz, dx],
                             preferred_element_type=jnp.float32)

    return lax.fori_loop(0, n_taps, tap_body, jnp.zeros((m, n), jnp.float32))


def _store_padded(out_ref, val5, zo, xo, zr=2):
    z = jnp.float32(0.0)
    out_ref[pl.ds(0, 1)] = jnp.broadcast_to(z, out_ref.shape)[0:1]
    out_ref[pl.ds(zo + 1, zr)] = jnp.broadcast_to(z, out_ref.shape)[:zr]
    out_ref[:, pl.ds(0, 1)] = jnp.broadcast_to(z, out_ref.shape)[:, 0:1]
    out_ref[:, pl.ds(_YB + 1, 1)] = jnp.broadcast_to(z, out_ref.shape)[:, 0:1]
    out_ref[:, :, pl.ds(0, 1)] = jnp.broadcast_to(z, out_ref.shape)[:, :, 0:1]
    out_ref[:, :, pl.ds(xo + 1, 1)] = jnp.broadcast_to(
        z, out_ref.shape)[:, :, 0:1]
    out_ref[pl.ds(1, zo), pl.ds(1, _YB), pl.ds(1, xo), :] = val5


def _conv_core(src_ref, wf_ref, sa, sz, sx, zo, xo, bz=0, bx=0):
    _, tz, tx, k, n = wf_ref.shape
    m = zo * _YB * xo
    return _taps(src_ref, wf_ref, sa, sz, sx, zo, xo, bz, bx,
                 m, k, n, tz, tx)


def _interior(ref, zo, xo):
    return ref[pl.ds(1, zo), pl.ds(1, _YB), pl.ds(1, xo), :]


def _level1(xp, maskp, w_in_f, sa_in, wblk, sa1):
    zo, xo = 25, 64
    m = zo * _YB * xo

    def body(xp_ref, mask_ref, win_ref, wa1_ref, wa2_ref, wb1_ref, wb2_ref,
             out_ref, scr_ref):
        mi = mask_ref[...].reshape(m, 128)

        acc = _conv_core(xp_ref, win_ref, sa_in, 1, 1, zo, xo)
        _store_padded(out_ref, jnp.maximum(acc * mi * _BNS, 0.0)
                      .reshape(zo, _YB, xo, 128), zo, xo)
        for wc1, wc2 in ((wa1_ref, wa2_ref), (wb1_ref, wb2_ref)):
            acc = _conv_core(out_ref, wc1, sa1, 1, 1, zo, xo)
            _store_padded(scr_ref, jnp.maximum(acc * mi * _BNS, 0.0)
                          .reshape(zo, _YB, xo, 128), zo, xo, zr=1)
            acc = _conv_core(scr_ref, wc2, sa1, 1, 1, zo, xo)
            ident = _interior(out_ref, zo, xo).reshape(m, 128)
            _store_padded(out_ref,
                          jnp.maximum(acc * mi * _BNS + ident, 0.0)
                          .reshape(zo, _YB, xo, 128), zo, xo)

    shape = (zo + 3, _YB + 2, xo + 2, 128)
    sshape = (zo + 2, _YB + 2, xo + 2, 128)
    return _pcall(body, jax.ShapeDtypeStruct(shape, jnp.float32),
                  scratch_shapes=[pltpu.VMEM(sshape, jnp.float32)])(
                      xp, maskp, w_in_f, *wblk)


def _levels234(h1, maskp, wd, dil, wblk, wout, sa_out):
    dims = [(13, 32), (7, 16), (3, 8)]

    def body(*refs):
        (h1_ref, m1_ref, wd2_ref, wd3_ref, wd4_ref, dl2_ref, dl3_ref,
         dl4_ref, wa21, wa22, wb21, wb22, wa31, wa32, wb31, wb32,
         wa41, wa42, wb41, wb42, wout_ref, out_ref,
         h2_ref, s2_ref, m2_ref, h3_ref, s3_ref, m3_ref,
         h4_ref, s4_ref, m4_ref) = refs

        lvl = [
            (h1_ref, m1_ref, wd2_ref, dl2_ref, (wa21, wa22, wb21, wb22),
             h2_ref, s2_ref, m2_ref, dims[0], 0),
            (h2_ref, m2_ref, wd3_ref, dl3_ref, (wa31, wa32, wb31, wb32),
             h3_ref, s3_ref, m3_ref, dims[1], 0),
            (h3_ref, m3_ref, wd4_ref, dl4_ref, (wa41, wa42, wb41, wb42),
             h4_ref, s4_ref, m4_ref, dims[2], 1),
        ]
        for (hin, min_, wd_ref, dl_ref, (w1, w2, w3, w4), hout, scr, mout,
             (zo, xo), bz) in lvl:
            m = zo * _YB * xo
            acc = _conv_core(hin, wd_ref, [0, 1], 2, 2, zo, xo, bz=bz)
            _store_padded(hout, jnp.maximum(acc * _BNS, 0.0)
                          .reshape(zo, _YB, xo, 128), zo, xo)
            macc = _conv_core(min_, dl_ref, [0, 1], 2, 2, zo, xo, bz=bz)
            _store_padded(mout, (macc > 0.0).astype(jnp.float32)
                          .reshape(zo, _YB, xo, 128), zo, xo)
            mi = _interior(mout, zo, xo).reshape(m, 128)
            for wc1, wc2 in ((w1, w2), (w3, w4)):
                acc = _conv_core(hout, wc1, [0, 1, 2], 1, 1, zo, xo)
                _store_padded(scr, jnp.maximum(acc * mi * _BNS, 0.0)
                              .reshape(zo, _YB, xo, 128), zo, xo)
                acc = _conv_core(scr, wc2, [0, 1, 2], 1, 1, zo, xo)
                ident = _interior(hout, zo, xo).reshape(m, 128)
                _store_padded(hout,
                              jnp.maximum(acc * mi * _BNS + ident, 0.0)
                              .reshape(zo, _YB, xo, 128), zo, xo)

        acc = _conv_core(h4_ref, wout_ref, sa_out, 2, 1, 1, 8, bz=1, bx=1)
        out_ref[...] = jnp.maximum(acc * _BNS, 0.0).reshape(1, _YB, 8, 128)

    def pbuf(zo, xo):
        return (zo + 3, _YB + 2, xo + 2, 128)

    scratch = []
    for zo, xo in dims:
        scratch += [pltpu.VMEM(pbuf(zo, xo), jnp.float32)] * 3
    return _pcall(body, jax.ShapeDtypeStruct((1, _YB, 8, 128), jnp.float32),
                  scratch_shapes=scratch)(
                      h1, maskp, wd[0], wd[1], wd[2], dil[0], dil[1], dil[2],
                      *wblk, wout)


def kernel(voxel_features, voxel_coords, batch_size, params):
    Z, Y, X = _SPATIAL
    N = voxel_features.shape[0]
    p = params

    b = voxel_coords[:, 0] % batch_size
    z = voxel_coords[:, 1] % Z
    y = voxel_coords[:, 2] % Y
    x = voxel_coords[:, 3] % X

    feats16 = jnp.concatenate(
        [voxel_features,
         jnp.ones((N, 1), jnp.float32),
         jnp.zeros((N, _DR - 6), jnp.float32)], axis=1)
    idx = ((((b * Z + z) * _YB + y // 8) * X + x) * 8 + y % 8).astype(jnp.int32)
    npad = (-N) % 1024
    pad_i = jnp.full((npad,), _NSH, jnp.int32)
    idx0 = jnp.concatenate(
        [jnp.where(idx < _NSH, idx, _NSH).astype(jnp.int32), pad_i])
    idx1 = jnp.concatenate(
        [jnp.where(idx >= _NSH, idx - _NSH, _NSH).astype(jnp.int32), pad_i])
    feats_p = jnp.concatenate([feats16, jnp.zeros((npad, _DR), jnp.float32)])
    table = _sc_scatter(idx0, idx1, feats_p)
    dense8f = table.reshape(_B, Z, _YB, X, 128)
    mask1x = jnp.repeat(dense8f[..., 5::16], 16, axis=-1)
    mask1xp = jnp.pad(mask1x, ((0, 0), (1, 2), (1, 1), (1, 1), (0, 0)))

    w_in16 = jnp.pad(p['w_in'], ((0, 0), (0, 0), (0, 0), (0, 11), (0, 0)))
    w_in_f, sa_in = _fold_w(w_in16, 8, 8, 1, 1)

    wblk1 = []
    sa1 = None
    for k_, g, ci in [('r1a1', 8, 16), ('r1a2', 8, 16), ('r1b1', 8, 16),
                      ('r1b2', 8, 16)]:
        wf, sa1 = _fold_w(p[k_], g, g, 1, 1)
        wblk1.append(wf)
    wblk234 = []
    for k_, g, ci in [('r2a1', 4, 32), ('r2a2', 4, 32), ('r2b1', 4, 32),
                      ('r2b2', 4, 32),
                      ('r3a1', 2, 64), ('r3a2', 2, 64), ('r3b1', 2, 64),
                      ('r3b2', 2, 64),
                      ('r4a1', 1, 128), ('r4a2', 1, 128), ('r4b1', 1, 128),
                      ('r4b2', 1, 128)]:
        wf, _ = _fold_w(p[k_], g, g, 1, 1)
        wblk234.append(wf)
    wd2, _ = _fold_w(p['w_d2'], 8, 4, 2, 1)
    wd3, _ = _fold_w(p['w_d3'], 4, 2, 2, 1)
    wd4, _ = _fold_w(p['w_d4'], 2, 1, 2, 1)
    wout, sa_out = _fold_w(p['w_out'], 1, 1, 1, 0)
    dil2, _ = _fold_w(jnp.ones((3, 3, 3, 16, 32), jnp.float32), 8, 4, 2, 1)
    dil3, _ = _fold_w(jnp.ones((3, 3, 3, 32, 64), jnp.float32), 4, 2, 2, 1)
    dil4, _ = _fold_w(jnp.ones((3, 3, 3, 64, 128), jnp.float32), 2, 1, 2, 1)

    outs = []
    for bi in range(_B):
        xp0 = jnp.pad(dense8f[bi], ((1, 2), (1, 1), (1, 1), (0, 0)))
        h1 = _level1(xp0, mask1x[bi], w_in_f, sa_in, wblk1, sa1)
        out = _levels234(h1, mask1xp[bi], (wd2, wd3, wd4),
                         (dil2, dil3, dil4), wblk234, wout, sa_out)
        outs.append(out)

    return jnp.stack(outs)


# speedup vs baseline: 1.1967x; 1.1874x over previous
import functools
import math

import numpy as np

import jax
import jax.numpy as jnp
from jax import lax
from jax.experimental import pallas as pl
from jax.experimental.pallas import tpu as pltpu
from jax.experimental.pallas import tpu_sc as plsc

_BNS = 1.0 / math.sqrt(1.0 + 1e-3)
_SPATIAL = (25, 64, 64)
_B = 2
_YB = 8


def _pcall(body, out_shape, scratch_shapes=(), interpret=False):
    return pl.pallas_call(body, out_shape=out_shape,
                          scratch_shapes=list(scratch_shapes),
                          interpret=interpret)


_NSITE = _B * 25 * 64 * 64
_TRASH = 128
_DR = 16
_NSH = _NSITE // 2
_HALF = _NSH + _TRASH


def _sc_scatter(idx0, idx1, feats16):
    NP = idx0.shape[0]
    ZB = 128
    CH = 512
    GR = 128
    n_chunks = NP // CH
    n_groups = CH // GR
    rows_t = _NSH // 16
    nz_full, nz_rem = divmod(rows_t, ZB)

    mesh = plsc.VectorSubcoreMesh(core_axis_name="c", subcore_axis_name="s")
    scratch = ([pltpu.VMEM_SHARED((_HALF, _DR), jnp.float32),
                pltpu.VMEM((ZB, _DR), jnp.float32)]
               + [pltpu.VMEM((CH,), jnp.int32) for _ in range(2)]
               + [pltpu.VMEM((CH, _DR), jnp.float32) for _ in range(2)]
               + [pltpu.SemaphoreType.DMA for _ in range(4)])

    @functools.partial(pl.kernel, mesh=mesh,
                       out_type=jax.ShapeDtypeStruct((2 * _NSH, _DR),
                                                     jnp.float32),
                       scratch_types=scratch,
                       compiler_params=pltpu.CompilerParams(
                           use_tc_tiling_on_sc=False))
    def run(idx0_hbm, idx1_hbm, feats_hbm, out_hbm, shared, zbuf,
            ib0, ib1, rv0, rv1, zsem, l0sem, l1sem, ssem):
        idx_bufs = (ib0, ib1)
        rows_bufs = (rv0, rv1)
        load_sems = (l0sem, l1sem)
        cid = lax.axis_index("c")
        sid = lax.axis_index("s")

        def zrow(r, carry):
            zbuf[r] = jnp.zeros((_DR,), jnp.float32)
            return carry
        lax.fori_loop(0, ZB, zrow, 0)
        base = sid * rows_t
        zcopies = []
        for k in range(nz_full):
            zcopies.append(pltpu.async_copy(
                zbuf, shared.at[pl.ds(base + k * ZB, ZB)], zsem))
        if nz_rem:
            zcopies.append(pltpu.async_copy(
                zbuf.at[pl.ds(0, nz_rem)],
                shared.at[pl.ds(base + nz_full * ZB, nz_rem)], zsem))
        for cp in zcopies:
            cp.wait()
        plsc.subcore_barrier()

        for c in range(2):
            @pl.when(jnp.logical_and(cid == c, sid == 0))
            def _scatter_phase(c=c):
                ih = idx0_hbm if c == 0 else idx1_hbm

                def load(ch, sl):
                    return (pltpu.async_copy(
                                feats_hbm.at[pl.ds(ch * CH, CH)],
                                rows_bufs[sl], load_sems[sl]),
                            pltpu.async_copy(
                                ih.at[pl.ds(ch * CH, CH)],
                                idx_bufs[sl], load_sems[sl]))

                pend = load(0, 0)
                for ch in range(n_chunks):
                    sl = ch % 2
                    for cp in pend:
                        cp.wait()
                    if ch + 1 < n_chunks:
                        pend = load(ch + 1, 1 - sl)
                    pltpu.async_copy(rows_bufs[sl],
                                     shared.at[idx_bufs[sl]], ssem).wait()
        plsc.subcore_barrier()
        pltpu.sync_copy(shared.at[pl.ds(base, rows_t)],
                        out_hbm.at[pl.ds(cid * _NSH + base, rows_t)])

    return run(idx0, idx1, feats16)


def _fold_w(w, gi, go, sy, py, scale=1.0):
    tz, ty, tx, ci, co = w.shape
    P = np.zeros((3, gi, go, ty), np.float32)
    for si, s in enumerate((-1, 0, 1)):
        for g_in in range(gi):
            for g_out in range(go):
                dy = g_in + s * gi - sy * g_out + py
                if 0 <= dy < ty:
                    P[si, g_in, g_out, dy] = scale
    wf = jnp.einsum('sghy,zyxio->szxgiho', P, w)
    wf = wf.reshape(3, tz, tx, gi * ci, go * co).astype(jnp.bfloat16)
    s_active = [si for si in range(3) if P[si].any()]
    return wf, s_active


def _taps(xp_ref, wf_ref, s_active, sz, sx, zo, xo, bz, bx, m, k, n, tz, tx):
    n_outer = len(s_active) * tz
    s0 = s_active[0]

    def tap_body(t, acc):
        j = t // tz
        dz = t % tz
        si = j + s0
        for dx in range(tx):
            xs = xp_ref[pl.ds(dz + bz, sz * zo), pl.ds(si, _YB),
                        pl.ds(dx + bx, sx * xo), :]
            if sz > 1 or sx > 1:
                xs = xs.reshape(zo, sz, _YB, xo, sx, k)[:, 0, :, :, 0, :]
            acc = acc + jnp.dot(xs.reshape(m, k).astype(jnp.bfloat16),
                                wf_ref[si, dz, dx],
                                preferred_element_type=jnp.float32)
        return acc

    return lax.fori_loop(0, n_outer, tap_body, jnp.zeros((m, n), jnp.float32))


def _store_padded(out_ref, val5, zo, xo, zr=2):
    z = jnp.zeros((), out_ref.dtype)
    out_ref[pl.ds(0, 1)] = jnp.broadcast_to(z, out_ref.shape)[0:1]
    out_ref[pl.ds(zo + 1, zr)] = jnp.broadcast_to(z, out_ref.shape)[:zr]
    out_ref[:, pl.ds(0, 1)] = jnp.broadcast_to(z, out_ref.shape)[:, 0:1]
    out_ref[:, pl.ds(_YB + 1, 1)] = jnp.broadcast_to(z, out_ref.shape)[:, 0:1]
    out_ref[:, :, pl.ds(0, 1)] = jnp.broadcast_to(z, out_ref.shape)[:, :, 0:1]
    out_ref[:, :, pl.ds(xo + 1, 1)] = jnp.broadcast_to(
        z, out_ref.shape)[:, :, 0:1]
    out_ref[pl.ds(1, zo), pl.ds(1, _YB), pl.ds(1, xo), :] = val5


def _conv_core(src_ref, wf_ref, sa, sz, sx, zo, xo, bz=0, bx=0):
    _, tz, tx, k, n = wf_ref.shape
    m = zo * _YB * xo
    return _taps(src_ref, wf_ref, sa, sz, sx, zo, xo, bz, bx,
                 m, k, n, tz, tx)


def _interior(ref, zo, xo):
    return ref[pl.ds(1, zo), pl.ds(1, _YB), pl.ds(1, xo), :]


def _level1(xp, maskp, w_in_f, sa_in, wblk, sa1):
    zo, xo = 25, 64
    m = zo * _YB * xo

    def body(xp_ref, mask_ref, win_ref, wa1_ref, wa2_ref, wb1_ref, wb2_ref,
             out_ref, hq_ref, sq_ref):
        mi = mask_ref[...].reshape(m, 128)

        acc = _conv_core(xp_ref, win_ref, sa_in, 1, 1, zo, xo)
        val = jnp.maximum(acc * mi * _BNS, 0.0).reshape(zo, _YB, xo, 128)
        _store_padded(out_ref, val, zo, xo)
        _store_padded(hq_ref, val.astype(jnp.bfloat16), zo, xo, zr=1)
        for wc1, wc2 in ((wa1_ref, wa2_ref), (wb1_ref, wb2_ref)):
            acc = _conv_core(hq_ref, wc1, sa1, 1, 1, zo, xo)
            _store_padded(sq_ref, jnp.maximum(acc * mi * _BNS, 0.0)
                          .reshape(zo, _YB, xo, 128)
                          .astype(jnp.bfloat16), zo, xo, zr=1)
            acc = _conv_core(sq_ref, wc2, sa1, 1, 1, zo, xo)
            ident = _interior(out_ref, zo, xo).reshape(m, 128)
            val = jnp.maximum(acc * mi * _BNS + ident, 0.0
                              ).reshape(zo, _YB, xo, 128)
            _store_padded(out_ref, val, zo, xo)
            _store_padded(hq_ref, val.astype(jnp.bfloat16), zo, xo, zr=1)

    shape = (zo + 3, _YB + 2, xo + 2, 128)
    sshape = (zo + 2, _YB + 2, xo + 2, 128)
    return _pcall(body, jax.ShapeDtypeStruct(shape, jnp.float32),
                  scratch_shapes=[pltpu.VMEM(sshape, jnp.bfloat16),
                                  pltpu.VMEM(sshape, jnp.bfloat16)])(
                      xp, maskp, w_in_f, *wblk)


def _levels234(h1, maskp, wd, dil, wblk, wout, sa_out):
    dims = [(13, 32), (7, 16), (3, 8)]

    def body(*refs):
        (h1_ref, m1_ref, wd2_ref, wd3_ref, wd4_ref, dl2_ref, dl3_ref,
         dl4_ref, wa21, wa22, wb21, wb22, wa31, wa32, wb31, wb32,
         wa41, wa42, wb41, wb42, wout_ref, out_ref,
         h2_ref, s2_ref, m2_ref, h3_ref, s3_ref, m3_ref,
         h4_ref, s4_ref, m4_ref) = refs

        lvl = [
            (h1_ref, m1_ref, wd2_ref, dl2_ref, (wa21, wa22, wb21, wb22),
             h2_ref, s2_ref, m2_ref, dims[0], 0),
            (h2_ref, m2_ref, wd3_ref, dl3_ref, (wa31, wa32, wb31, wb32),
             h3_ref, s3_ref, m3_ref, dims[1], 0),
            (h3_ref, m3_ref, wd4_ref, dl4_ref, (wa41, wa42, wb41, wb42),
             h4_ref, s4_ref, m4_ref, dims[2], 1),
        ]
        for (hin, min_, wd_ref, dl_ref, (w1, w2, w3, w4), hout, scr, mout,
             (zo, xo), bz) in lvl:
            m = zo * _YB * xo
            acc = _conv_core(hin, wd_ref, [0, 1], 2, 2, zo, xo, bz=bz)
            _store_padded(hout, jnp.maximum(acc * _BNS, 0.0)
                          .reshape(zo, _YB, xo, 128), zo, xo)
            macc = _conv_core(min_, dl_ref, [0, 1], 2, 2, zo, xo, bz=bz)
            _store_padded(mout, (macc > 0.0).astype(jnp.float32)
                          .reshape(zo, _YB, xo, 128), zo, xo)
            mi = _interior(mout, zo, xo).reshape(m, 128)
            for wc1, wc2 in ((w1, w2), (w3, w4)):
                acc = _conv_core(hout, wc1, [0, 1, 2], 1, 1, zo, xo)
                _store_padded(scr, jnp.maximum(acc * mi * _BNS, 0.0)
                              .reshape(zo, _YB, xo, 128), zo, xo)
                acc = _conv_core(scr, wc2, [0, 1, 2], 1, 1, zo, xo)
                ident = _interior(hout, zo, xo).reshape(m, 128)
                _store_padded(hout,
                              jnp.maximum(acc * mi * _BNS + ident, 0.0)
                              .reshape(zo, _YB, xo, 128), zo, xo)

        acc = _conv_core(h4_ref, wout_ref, sa_out, 2, 1, 1, 8, bz=1, bx=1)
        out_ref[...] = jnp.maximum(acc * _BNS, 0.0).reshape(1, _YB, 8, 128)

    def pbuf(zo, xo):
        return (zo + 3, _YB + 2, xo + 2, 128)

    scratch = []
    for zo, xo in dims:
        scratch += [pltpu.VMEM(pbuf(zo, xo), jnp.float32)] * 3
    return _pcall(body, jax.ShapeDtypeStruct((1, _YB, 8, 128), jnp.float32),
                  scratch_shapes=scratch)(
                      h1, maskp, wd[0], wd[1], wd[2], dil[0], dil[1], dil[2],
                      *wblk, wout)


def kernel(voxel_features, voxel_coords, batch_size, params):
    Z, Y, X = _SPATIAL
    N = voxel_features.shape[0]
    p = params

    b = voxel_coords[:, 0] % batch_size
    z = voxel_coords[:, 1] % Z
    y = voxel_coords[:, 2] % Y
    x = voxel_coords[:, 3] % X

    feats16 = jnp.concatenate(
        [voxel_features,
         jnp.ones((N, 1), jnp.float32),
         jnp.zeros((N, _DR - 6), jnp.float32)], axis=1)
    idx = ((((b * Z + z) * _YB + y // 8) * X + x) * 8 + y % 8).astype(jnp.int32)
    npad = (-N) % 1024
    pad_i = jnp.full((npad,), _NSH, jnp.int32)
    idx0 = jnp.concatenate(
        [jnp.where(idx < _NSH, idx, _NSH).astype(jnp.int32), pad_i])
    idx1 = jnp.concatenate(
        [jnp.where(idx >= _NSH, idx - _NSH, _NSH).astype(jnp.int32), pad_i])
    feats_p = jnp.concatenate([feats16, jnp.zeros((npad, _DR), jnp.float32)])
    table = _sc_scatter(idx0, idx1, feats_p)
    dense8f = table.reshape(_B, Z, _YB, X, 128)
    mask1x = jnp.repeat(dense8f[..., 5::16], 16, axis=-1)
    mask1xp = jnp.pad(mask1x, ((0, 0), (1, 2), (1, 1), (1, 1), (0, 0)))

    w_in16 = jnp.pad(p['w_in'], ((0, 0), (0, 0), (0, 0), (0, 11), (0, 0)))
    w_in_f, sa_in = _fold_w(w_in16, 8, 8, 1, 1)

    wblk1 = []
    sa1 = None
    for k_, g, ci in [('r1a1', 8, 16), ('r1a2', 8, 16), ('r1b1', 8, 16),
                      ('r1b2', 8, 16)]:
        wf, sa1 = _fold_w(p[k_], g, g, 1, 1)
        wblk1.append(wf)
    wblk234 = []
    for k_, g, ci in [('r2a1', 4, 32), ('r2a2', 4, 32), ('r2b1', 4, 32),
                      ('r2b2', 4, 32),
                      ('r3a1', 2, 64), ('r3a2', 2, 64), ('r3b1', 2, 64),
                      ('r3b2', 2, 64),
                      ('r4a1', 1, 128), ('r4a2', 1, 128), ('r4b1', 1, 128),
                      ('r4b2', 1, 128)]:
        wf, _ = _fold_w(p[k_], g, g, 1, 1)
        wblk234.append(wf)
    wd2, _ = _fold_w(p['w_d2'], 8, 4, 2, 1)
    wd3, _ = _fold_w(p['w_d3'], 4, 2, 2, 1)
    wd4, _ = _fold_w(p['w_d4'], 2, 1, 2, 1)
    wout, sa_out = _fold_w(p['w_out'], 1, 1, 1, 0)
    dil2, _ = _fold_w(jnp.ones((3, 3, 3, 16, 32), jnp.float32), 8, 4, 2, 1)
    dil3, _ = _fold_w(jnp.ones((3, 3, 3, 32, 64), jnp.float32), 4, 2, 2, 1)
    dil4, _ = _fold_w(jnp.ones((3, 3, 3, 64, 128), jnp.float32), 2, 1, 2, 1)

    outs = []
    for bi in range(_B):
        xp0 = jnp.pad(dense8f[bi], ((1, 2), (1, 1), (1, 1), (0, 0)))
        h1 = _level1(xp0, mask1x[bi].astype(jnp.bfloat16), w_in_f, sa_in,
                     wblk1, sa1)
        out = _levels234(h1, mask1xp[bi], (wd2, wd3, wd4),
                         (dil2, dil3, dil4), wblk234, wout, sa_out)
        outs.append(out)

    return jnp.stack(outs)
